# baseline (speedup 1.0000x reference)
import functools

import jax
import jax.numpy as jnp
from jax import lax
from jax.experimental import pallas as pl
from jax.experimental.pallas import tpu as pltpu
from jax.experimental.pallas import tpu_sc as plsc

VOCAB = 100000
EMBED_DIM = 64
BATCH = 4096
SEQ = 200

NUM_CORES = 2
NUM_SUBCORES = 16
NUM_WORKERS = NUM_CORES * NUM_SUBCORES
ROWS_PER_WORKER = BATCH // NUM_WORKERS
IDX_PER_WORKER = ROWS_PER_WORKER * SEQ
CHUNK_A = 128
CHUNK_B = SEQ - CHUNK_A

_LANES = 16
_PACKED_DIM = EMBED_DIM // 2
_GROUPS = EMBED_DIM // (2 * _LANES)


def _pack_table(embed_weight):
    t = embed_weight.astype(jnp.bfloat16)
    t = t.reshape(VOCAB, _GROUPS, 2, _LANES).transpose(0, 1, 3, 2)
    return jax.lax.bitcast_convert_type(
        t.reshape(VOCAB, _PACKED_DIM, 2), jnp.int32
    )


def _sc_mean_pool(idx_flat, table_packed):
    mesh = plsc.VectorSubcoreMesh(core_axis_name="c", subcore_axis_name="s")

    @functools.partial(
        pl.kernel,
        mesh=mesh,
        out_type=jax.ShapeDtypeStruct((BATCH, EMBED_DIM), jnp.float32),
        compiler_params=pltpu.CompilerParams(
            use_tc_tiling_on_sc=False, needs_layout_passes=False
        ),
        scratch_types=[
            pltpu.VMEM((IDX_PER_WORKER,), jnp.int32),
            pltpu.VMEM((SEQ, _PACKED_DIM), jnp.int32),
            pltpu.VMEM((SEQ, _PACKED_DIM), jnp.int32),
            pltpu.VMEM((SEQ, _PACKED_DIM), jnp.int32),
            pltpu.VMEM((SEQ, _PACKED_DIM), jnp.int32),
            pltpu.VMEM((ROWS_PER_WORKER, EMBED_DIM), jnp.float32),
            pltpu.SemaphoreType.DMA,
            pltpu.SemaphoreType.DMA,
            pltpu.SemaphoreType.DMA,
            pltpu.SemaphoreType.DMA,
        ],
    )
    def k(
        table_hbm,
        idx_hbm,
        out_hbm,
        idx_v,
        buf0,
        buf1,
        buf2,
        buf3,
        stage_v,
        sem0,
        sem1,
        sem2,
        sem3,
    ):
        wid = lax.axis_index("s") * NUM_CORES + lax.axis_index("c")
        base = wid * IDX_PER_WORKER
        pltpu.sync_copy(idx_hbm.at[pl.ds(base, IDX_PER_WORKER)], idx_v)

        def issue(r, buf, sem):
            off = r * SEQ
            pltpu.async_copy(
                table_hbm.at[idx_v.at[pl.ds(off, CHUNK_A)]],
                buf.at[pl.ds(0, CHUNK_A)],
                sem,
            )
            pltpu.async_copy(
                table_hbm.at[idx_v.at[pl.ds(off + CHUNK_A, CHUNK_B)]],
                buf.at[pl.ds(CHUNK_A, CHUNK_B)],
                sem,
            )

        def wait(buf, sem):
            pltpu.make_async_copy(table_hbm.at[pl.ds(0, SEQ)], buf, sem).wait()

        hi_mask = jnp.full((_LANES,), -65536, jnp.int32)

        UNROLL = 25

        def reduce_store(r, buf):
            zeros = jnp.zeros((_LANES,), jnp.float32)

            def body(i, accs):
                new = list(accs)
                for u in range(UNROLL):
                    l = i * UNROLL + u
                    for g in range(_GROUPS):
                        w = buf[l, pl.ds(g * _LANES, _LANES)]
                        lo = plsc.bitcast(lax.shift_left(w, 16), jnp.float32)
                        hi = plsc.bitcast(
                            lax.bitwise_and(w, hi_mask), jnp.float32
                        )
                        new[2 * g] = new[2 * g] + lo
                        new[2 * g + 1] = new[2 * g + 1] + hi
                return tuple(new)

            accs = lax.fori_loop(0, SEQ // UNROLL, body, (zeros,) * (2 * _GROUPS))
            scale = jnp.float32(1.0 / SEQ)
            for d in range(2 * _GROUPS):
                stage_v[r, pl.ds(d * _LANES, _LANES)] = accs[d] * scale

        bufs = (buf0, buf1, buf2, buf3)
        sems = (sem0, sem1, sem2, sem3)
        NBUF = 4

        for j in range(NBUF - 1):
            issue(j, bufs[j], sems[j])

        @pl.loop(0, ROWS_PER_WORKER, step=NBUF)
        def _(g):
            for j in range(NBUF):
                r = g + j
                pf = (j + NBUF - 1) % NBUF

                @pl.when(r + NBUF - 1 < ROWS_PER_WORKER)
                def _():
                    issue(r + NBUF - 1, bufs[pf], sems[pf])

                wait(bufs[j], sems[j])
                reduce_store(r, bufs[j])

        pltpu.sync_copy(
            stage_v, out_hbm.at[pl.ds(wid * ROWS_PER_WORKER, ROWS_PER_WORKER)]
        )

    return k(table_packed, idx_flat)


def kernel(input_ids, embed_weight):
    idx_flat = input_ids.reshape(-1)
    return _sc_mean_pool(idx_flat, _pack_table(embed_weight))


# speedup vs baseline: 1.0137x; 1.0137x over previous
import functools

import jax
import jax.numpy as jnp
from jax import lax
from jax.experimental import pallas as pl
from jax.experimental.pallas import tpu as pltpu
from jax.experimental.pallas import tpu_sc as plsc

VOCAB = 100000
EMBED_DIM = 64
BATCH = 4096
SEQ = 200

NUM_CORES = 2
NUM_SUBCORES = 16
NUM_WORKERS = NUM_CORES * NUM_SUBCORES
ROWS_PER_WORKER = BATCH // NUM_WORKERS
IDX_PER_WORKER = ROWS_PER_WORKER * SEQ
CHUNK_A = 128
CHUNK_B = SEQ - CHUNK_A

_LANES = 16
_PACKED_DIM = EMBED_DIM // 2
_GROUPS = EMBED_DIM // (2 * _LANES)


def _pack_table(embed_weight):
    t = embed_weight.astype(jnp.bfloat16)
    t = t.reshape(VOCAB, _GROUPS, 2, _LANES).transpose(0, 1, 3, 2)
    return jax.lax.bitcast_convert_type(
        t.reshape(VOCAB, _PACKED_DIM, 2), jnp.int32
    )


def _sc_mean_pool(idx_flat, table_packed):
    mesh = plsc.VectorSubcoreMesh(core_axis_name="c", subcore_axis_name="s")

    @functools.partial(
        pl.kernel,
        mesh=mesh,
        out_type=jax.ShapeDtypeStruct((BATCH, EMBED_DIM), jnp.float32),
        compiler_params=pltpu.CompilerParams(
            use_tc_tiling_on_sc=False, needs_layout_passes=False
        ),
        scratch_types=[
            pltpu.VMEM((IDX_PER_WORKER,), jnp.int32),
            pltpu.VMEM((SEQ, _PACKED_DIM), jnp.int32),
            pltpu.VMEM((SEQ, _PACKED_DIM), jnp.int32),
            pltpu.VMEM((SEQ, _PACKED_DIM), jnp.int32),
            pltpu.VMEM((SEQ, _PACKED_DIM), jnp.int32),
            pltpu.VMEM((SEQ, _PACKED_DIM), jnp.int32),
            pltpu.VMEM((SEQ, _PACKED_DIM), jnp.int32),
            pltpu.VMEM((SEQ, _PACKED_DIM), jnp.int32),
            pltpu.VMEM((SEQ, _PACKED_DIM), jnp.int32),
            pltpu.VMEM((ROWS_PER_WORKER, EMBED_DIM), jnp.float32),
            pltpu.SemaphoreType.DMA,
            pltpu.SemaphoreType.DMA,
            pltpu.SemaphoreType.DMA,
            pltpu.SemaphoreType.DMA,
            pltpu.SemaphoreType.DMA,
            pltpu.SemaphoreType.DMA,
            pltpu.SemaphoreType.DMA,
            pltpu.SemaphoreType.DMA,
        ],
    )
    def k(
        table_hbm,
        idx_hbm,
        out_hbm,
        idx_v,
        buf0,
        buf1,
        buf2,
        buf3,
        buf4,
        buf5,
        buf6,
        buf7,
        stage_v,
        sem0,
        sem1,
        sem2,
        sem3,
        sem4,
        sem5,
        sem6,
        sem7,
    ):
        wid = lax.axis_index("s") * NUM_CORES + lax.axis_index("c")
        base = wid * IDX_PER_WORKER
        pltpu.sync_copy(idx_hbm.at[pl.ds(base, IDX_PER_WORKER)], idx_v)

        def issue(r, buf, sem):
            off = r * SEQ
            pltpu.async_copy(
                table_hbm.at[idx_v.at[pl.ds(off, CHUNK_A)]],
                buf.at[pl.ds(0, CHUNK_A)],
                sem,
            )
            pltpu.async_copy(
                table_hbm.at[idx_v.at[pl.ds(off + CHUNK_A, CHUNK_B)]],
                buf.at[pl.ds(CHUNK_A, CHUNK_B)],
                sem,
            )

        def wait(buf, sem):
            pltpu.make_async_copy(table_hbm.at[pl.ds(0, SEQ)], buf, sem).wait()

        hi_mask = jnp.full((_LANES,), -65536, jnp.int32)

        UNROLL = 8

        def reduce_store(r, buf):
            zeros = jnp.zeros((_LANES,), jnp.float32)

            def body(i, accs):
                new = list(accs)
                for u in range(UNROLL):
                    l = i * UNROLL + u
                    for g in range(_GROUPS):
                        w = buf[l, pl.ds(g * _LANES, _LANES)]
                        lo = plsc.bitcast(lax.shift_left(w, 16), jnp.float32)
                        hi = plsc.bitcast(
                            lax.bitwise_and(w, hi_mask), jnp.float32
                        )
                        new[2 * g] = new[2 * g] + lo
                        new[2 * g + 1] = new[2 * g + 1] + hi
                return tuple(new)

            accs = lax.fori_loop(0, SEQ // UNROLL, body, (zeros,) * (2 * _GROUPS))
            scale = jnp.float32(1.0 / SEQ)
            for d in range(2 * _GROUPS):
                stage_v[r, pl.ds(d * _LANES, _LANES)] = accs[d] * scale

        bufs = (buf0, buf1, buf2, buf3, buf4, buf5, buf6, buf7)
        sems = (sem0, sem1, sem2, sem3, sem4, sem5, sem6, sem7)
        NBUF = 8

        for j in range(NBUF - 1):
            issue(j, bufs[j], sems[j])

        @pl.loop(0, ROWS_PER_WORKER, step=NBUF)
        def _(g):
            for j in range(NBUF):
                r = g + j
                pf = (j + NBUF - 1) % NBUF

                @pl.when(r + NBUF - 1 < ROWS_PER_WORKER)
                def _():
                    issue(r + NBUF - 1, bufs[pf], sems[pf])

                wait(bufs[j], sems[j])
                reduce_store(r, bufs[j])

        pltpu.sync_copy(
            stage_v, out_hbm.at[pl.ds(wid * ROWS_PER_WORKER, ROWS_PER_WORKER)]
        )

    return k(table_packed, idx_flat)


def kernel(input_ids, embed_weight):
    idx_flat = input_ids.reshape(-1)
    return _sc_mean_pool(idx_flat, _pack_table(embed_weight))
